# Initial kernel scaffold; baseline (speedup 1.0000x reference)
import jax
import jax.numpy as jnp
from jax.experimental import pallas as pl


def kernel(x, edge_index, eps, gin_W1, gin_b1, gin_W2, gin_b2, W1, b1, W2, b2, W3, b3):
    raise NotImplementedError("write your pallas kernel here")


# speedup vs baseline: 47.9718x; 47.9718x over previous
import functools

import jax
import jax.numpy as jnp
from jax import lax
from jax.experimental import pallas as pl
from jax.experimental.pallas import tpu as pltpu
from jax.experimental.pallas import tpu_sc as plsc

_N = 100000
_E = 6400000
_F = 4
_FP = 8
_NC = 2
_NS = 16
_NW = _NC * _NS
_CHUNK = 128
_WCH = 16
_CPW = 1568
_EP = _NW * _CPW * _CHUNK
_NP = _N + 96
_RPT = _NP // _NS


def _sc_segment_sum(xp, srcp, dstp, zrows):
  mesh = plsc.VectorSubcoreMesh(core_axis_name="c", subcore_axis_name="s")

  @functools.partial(
      pl.kernel,
      out_type=jax.ShapeDtypeStruct((_NC, _NP, _FP), jnp.float32),
      mesh=mesh,
      compiler_params=pltpu.CompilerParams(use_tc_tiling_on_sc=False),
      scratch_types=[
          pltpu.VMEM_SHARED((_NP, _FP), jnp.float32),
          pltpu.VMEM_SHARED((_NP, _FP), jnp.float32),
          pltpu.VMEM((_WCH, _CHUNK), jnp.int32),
          pltpu.VMEM((_WCH, _CHUNK), jnp.int32),
          pltpu.VMEM((_CHUNK, _FP), jnp.float32),
      ],
  )
  def seg(x_hbm, src_hbm, dst_hbm, z_hbm, out_hbm, x_sh, agg_sh, sidx, didx,
          rbuf):
    cid = lax.axis_index("c")
    sid = lax.axis_index("s")
    w = cid * _NS + sid
    r0 = sid * _RPT
    pltpu.sync_copy(x_hbm.at[pl.ds(r0, _RPT)], x_sh.at[pl.ds(r0, _RPT)])
    pltpu.sync_copy(z_hbm.at[pl.ds(r0, _RPT)], agg_sh.at[pl.ds(r0, _RPT)])
    plsc.subcore_barrier()

    chunk0 = w * _CPW

    def window(win, carry):
      base = chunk0 + win * _WCH
      pltpu.sync_copy(src_hbm.at[pl.ds(base, _WCH)], sidx)
      pltpu.sync_copy(dst_hbm.at[pl.ds(base, _WCH)], didx)

      def chunk(j, c):
        pltpu.sync_copy(x_sh.at[sidx.at[j]], rbuf)
        pltpu.sync_copy(rbuf, agg_sh.at[didx.at[j]], add=True)
        return c

      lax.fori_loop(0, _WCH, chunk, 0)
      return carry

    lax.fori_loop(0, _CPW // _WCH, window, 0)
    plsc.subcore_barrier()
    pltpu.sync_copy(agg_sh.at[pl.ds(r0, _RPT)],
                    out_hbm.at[cid].at[pl.ds(r0, _RPT)])

  return seg(xp, srcp, dstp, zrows)


_BM = 10000


def _mlp_body(x_ref, p0_ref, p1_ref, eps_ref, gw1_ref, gb1_ref, gw2_ref,
              gb2_ref, w1_ref, b1_ref, w2_ref, b2_ref, w3_ref, b3_ref, o_ref):
  h = (1.0 + eps_ref[0]) * x_ref[...] + p0_ref[...] + p1_ref[...]
  h = jnp.dot(h, gw1_ref[...], preferred_element_type=jnp.float32)
  h = jnp.maximum(h + gb1_ref[...], 0.0)
  h = jnp.dot(h, gw2_ref[...], preferred_element_type=jnp.float32)
  h = h + gb2_ref[...]
  h = jnp.dot(h, w1_ref[...], preferred_element_type=jnp.float32) + b1_ref[...]
  h = 1.0 / (1.0 + jnp.exp(-h))
  h = jnp.dot(h, w2_ref[...], preferred_element_type=jnp.float32) + b2_ref[...]
  h = 1.0 / (1.0 + jnp.exp(-h))
  h = jnp.dot(h, w3_ref[...], preferred_element_type=jnp.float32) + b3_ref[...]
  o_ref[...] = 1.0 / (1.0 + jnp.exp(-h))


def _tc_mlp(x, p0, p1, eps, gin_W1, gin_b1, gin_W2, gin_b2, W1, b1, W2, b2,
            W3, b3):
  grid = (_N // _BM,)
  row_spec = pl.BlockSpec((_BM, _F), lambda i: (i, 0))

  def full(a):
    return pl.BlockSpec(a.shape, lambda i: tuple(0 for _ in a.shape))

  biases = [gin_b1, gin_b2, b1, b2, b3]
  weights = [gin_W1, gin_W2, W1, W2, W3]
  w_gw1, w_gw2, w_w1, w_w2, w_w3 = weights
  b_gb1, b_gb2, b_b1, b_b2, b_b3 = [b.reshape(1, -1) for b in biases]
  eps1 = eps.reshape(1)

  return pl.pallas_call(
      _mlp_body,
      grid=grid,
      in_specs=[
          row_spec, row_spec, row_spec,
          pl.BlockSpec(memory_space=pltpu.SMEM),
          full(w_gw1), full(b_gb1), full(w_gw2), full(b_gb2),
          full(w_w1), full(b_b1), full(w_w2), full(b_b2),
          full(w_w3), full(b_b3),
      ],
      out_specs=pl.BlockSpec((_BM, 1), lambda i: (i, 0)),
      out_shape=jax.ShapeDtypeStruct((_N, 1), jnp.float32),
  )(x, p0, p1, eps1, w_gw1, b_gb1, w_gw2, b_gb2, w_w1, b_b1, w_w2, b_b2,
    w_w3, b_b3)


def kernel(x, edge_index, eps, gin_W1, gin_b1, gin_W2, gin_b2, W1, b1, W2, b2,
           W3, b3):
  pad = _EP - _E
  sink = (_N + (jnp.arange(pad, dtype=jnp.int32) % 96)).astype(jnp.int32)
  srcp = jnp.concatenate([edge_index[0], sink]).reshape(-1, _CHUNK)
  dstp = jnp.concatenate([edge_index[1], sink]).reshape(-1, _CHUNK)
  xp = jnp.zeros((_NP, _FP), jnp.float32).at[:_N, :_F].set(x)
  zrows = jnp.zeros((_NP, _FP), jnp.float32)
  partials = _sc_segment_sum(xp, srcp, dstp, zrows)
  p0 = partials[0, :_N, :_F]
  p1 = partials[1, :_N, :_F]
  return _tc_mlp(x, p0, p1, eps, gin_W1, gin_b1, gin_W2, gin_b2, W1, b1, W2,
                 b2, W3, b3)


# speedup vs baseline: 60.8368x; 1.2682x over previous
import functools

import jax
import jax.numpy as jnp
from jax import lax
from jax.experimental import pallas as pl
from jax.experimental.pallas import tpu as pltpu
from jax.experimental.pallas import tpu_sc as plsc

_N = 100000
_E = 6400000
_F = 4
_FP = 8
_NC = 2
_NS = 16
_NW = _NC * _NS
_CHUNK = 128
_WCH = 16
_CPW = 1568
_EP = _NW * _CPW * _CHUNK
_NP = _N + 96
_RPT = _NP // _NS


def _sc_segment_sum(xp, srcp, dstp, zrows):
  mesh = plsc.VectorSubcoreMesh(core_axis_name="c", subcore_axis_name="s")

  @functools.partial(
      pl.kernel,
      out_type=jax.ShapeDtypeStruct((_NC, _NP, _FP), jnp.float32),
      mesh=mesh,
      compiler_params=pltpu.CompilerParams(use_tc_tiling_on_sc=False),
      scratch_types=[
          pltpu.VMEM_SHARED((_NP, _FP), jnp.float32),
          pltpu.VMEM_SHARED((_NP, _FP), jnp.float32),
          pltpu.VMEM((_WCH, _CHUNK), jnp.int32),
          pltpu.VMEM((_WCH, _CHUNK), jnp.int32),
          pltpu.VMEM((_CHUNK, _FP), jnp.float32),
          pltpu.VMEM((_CHUNK, _FP), jnp.float32),
          pltpu.SemaphoreType.DMA,
          pltpu.SemaphoreType.DMA,
          pltpu.SemaphoreType.DMA,
      ],
  )
  def seg(x_hbm, src_hbm, dst_hbm, z_hbm, out_hbm, x_sh, agg_sh, sidx, didx,
          rb0, rb1, gs0, gs1, isem):
    cid = lax.axis_index("c")
    sid = lax.axis_index("s")
    w = cid * _NS + sid
    r0 = sid * _RPT
    pltpu.sync_copy(x_hbm.at[pl.ds(r0, _RPT)], x_sh.at[pl.ds(r0, _RPT)])
    pltpu.sync_copy(z_hbm.at[pl.ds(r0, _RPT)], agg_sh.at[pl.ds(r0, _RPT)])
    plsc.subcore_barrier()

    chunk0 = w * _CPW

    def window(win, carry):
      base = chunk0 + win * _WCH
      pltpu.async_copy(src_hbm.at[pl.ds(base, _WCH)], sidx, isem)
      pltpu.async_copy(dst_hbm.at[pl.ds(base, _WCH)], didx, isem)
      pltpu.make_async_copy(src_hbm.at[pl.ds(base, _WCH)], sidx, isem).wait()
      pltpu.make_async_copy(dst_hbm.at[pl.ds(base, _WCH)], didx, isem).wait()

      pltpu.async_copy(x_sh.at[sidx.at[0]], rb0, gs0)

      def pair(k, c):
        j = 2 * k
        pltpu.make_async_copy(x_sh.at[sidx.at[j]], rb0, gs0).wait()
        pltpu.async_copy(x_sh.at[sidx.at[j + 1]], rb1, gs1)
        pltpu.sync_copy(rb0, agg_sh.at[didx.at[j]], add=True)
        pltpu.make_async_copy(x_sh.at[sidx.at[j + 1]], rb1, gs1).wait()

        @pl.when(k < _WCH // 2 - 1)
        def _():
          pltpu.async_copy(x_sh.at[sidx.at[j + 2]], rb0, gs0)

        pltpu.sync_copy(rb1, agg_sh.at[didx.at[j + 1]], add=True)
        return c

      lax.fori_loop(0, _WCH // 2, pair, 0)
      return carry

    lax.fori_loop(0, _CPW // _WCH, window, 0)
    plsc.subcore_barrier()
    pltpu.sync_copy(agg_sh.at[pl.ds(r0, _RPT)],
                    out_hbm.at[cid].at[pl.ds(r0, _RPT)])

  return seg(xp, srcp, dstp, zrows)


_BM = 10000


def _mlp_body(x_ref, p0_ref, p1_ref, eps_ref, gw1_ref, gb1_ref, gw2_ref,
              gb2_ref, w1_ref, b1_ref, w2_ref, b2_ref, w3_ref, b3_ref, o_ref):
  h = (1.0 + eps_ref[0]) * x_ref[...] + p0_ref[...] + p1_ref[...]
  h = jnp.dot(h, gw1_ref[...], preferred_element_type=jnp.float32)
  h = jnp.maximum(h + gb1_ref[...], 0.0)
  h = jnp.dot(h, gw2_ref[...], preferred_element_type=jnp.float32)
  h = h + gb2_ref[...]
  h = jnp.dot(h, w1_ref[...], preferred_element_type=jnp.float32) + b1_ref[...]
  h = 1.0 / (1.0 + jnp.exp(-h))
  h = jnp.dot(h, w2_ref[...], preferred_element_type=jnp.float32) + b2_ref[...]
  h = 1.0 / (1.0 + jnp.exp(-h))
  h = jnp.dot(h, w3_ref[...], preferred_element_type=jnp.float32) + b3_ref[...]
  o_ref[...] = 1.0 / (1.0 + jnp.exp(-h))


def _tc_mlp(x, p0, p1, eps, gin_W1, gin_b1, gin_W2, gin_b2, W1, b1, W2, b2,
            W3, b3):
  grid = (_N // _BM,)
  row_spec = pl.BlockSpec((_BM, _F), lambda i: (i, 0))

  def full(a):
    return pl.BlockSpec(a.shape, lambda i: tuple(0 for _ in a.shape))

  biases = [gin_b1, gin_b2, b1, b2, b3]
  weights = [gin_W1, gin_W2, W1, W2, W3]
  w_gw1, w_gw2, w_w1, w_w2, w_w3 = weights
  b_gb1, b_gb2, b_b1, b_b2, b_b3 = [b.reshape(1, -1) for b in biases]
  eps1 = eps.reshape(1)

  return pl.pallas_call(
      _mlp_body,
      grid=grid,
      in_specs=[
          row_spec, row_spec, row_spec,
          pl.BlockSpec(memory_space=pltpu.SMEM),
          full(w_gw1), full(b_gb1), full(w_gw2), full(b_gb2),
          full(w_w1), full(b_b1), full(w_w2), full(b_b2),
          full(w_w3), full(b_b3),
      ],
      out_specs=pl.BlockSpec((_BM, 1), lambda i: (i, 0)),
      out_shape=jax.ShapeDtypeStruct((_N, 1), jnp.float32),
  )(x, p0, p1, eps1, w_gw1, b_gb1, w_gw2, b_gb2, w_w1, b_b1, w_w2, b_b2,
    w_w3, b_b3)


def kernel(x, edge_index, eps, gin_W1, gin_b1, gin_W2, gin_b2, W1, b1, W2, b2,
           W3, b3):
  pad = _EP - _E
  sink = (_N + (jnp.arange(pad, dtype=jnp.int32) % 96)).astype(jnp.int32)
  srcp = jnp.concatenate([edge_index[0], sink]).reshape(-1, _CHUNK)
  dstp = jnp.concatenate([edge_index[1], sink]).reshape(-1, _CHUNK)
  xp = jnp.zeros((_NP, _FP), jnp.float32).at[:_N, :_F].set(x)
  zrows = jnp.zeros((_NP, _FP), jnp.float32)
  partials = _sc_segment_sum(xp, srcp, dstp, zrows)
  p0 = partials[0, :_N, :_F]
  p1 = partials[1, :_N, :_F]
  return _tc_mlp(x, p0, p1, eps, gin_W1, gin_b1, gin_W2, gin_b2, W1, b1, W2,
                 b2, W3, b3)


# speedup vs baseline: 99.4730x; 1.6351x over previous
import functools

import jax
import jax.numpy as jnp
from jax import lax
from jax.experimental import pallas as pl
from jax.experimental.pallas import tpu as pltpu
from jax.experimental.pallas import tpu_sc as plsc

_N = 100000
_E = 6400000
_F = 4
_FP = 8
_NC = 2
_NS = 16
_NW = _NC * _NS
_CHUNK = 128
_WCH = 16
_NCH = _E // _CHUNK
_CPW = _NCH // _NW
_XTRA = _NCH % _NW
_FW = _CPW // _WCH
_NP = _N + 96
_RPT = _NP // _NS
_PK = 128 // _FP
_NR = _NP // _PK


def _sc_segment_sum(xp, srcp, dstp, zrows):
  mesh = plsc.VectorSubcoreMesh(core_axis_name="c", subcore_axis_name="s")

  @functools.partial(
      pl.kernel,
      out_type=jax.ShapeDtypeStruct((_NC, _NP, _FP), jnp.float32),
      mesh=mesh,
      compiler_params=pltpu.CompilerParams(use_tc_tiling_on_sc=False),
      scratch_types=[
          pltpu.VMEM_SHARED((_NP, _FP), jnp.float32),
          pltpu.VMEM_SHARED((_NP, _FP), jnp.float32),
          pltpu.VMEM((2 * _WCH, _CHUNK), jnp.int32),
          pltpu.VMEM((2 * _WCH, _CHUNK), jnp.int32),
          pltpu.VMEM((_CHUNK, _FP), jnp.float32),
          pltpu.VMEM((_CHUNK, _FP), jnp.float32),
          pltpu.SemaphoreType.DMA,
          pltpu.SemaphoreType.DMA,
          pltpu.SemaphoreType.DMA,
      ],
  )
  def seg(x_hbm, src_hbm, dst_hbm, z_hbm, out_hbm, x_sh, agg_sh, sidx, didx,
          rb0, rb1, gs0, gs1, isem):
    cid = lax.axis_index("c")
    sid = lax.axis_index("s")
    w = cid * _NS + sid
    r0 = sid * _RPT
    pltpu.sync_copy(x_hbm.at[pl.ds(r0, _RPT)], x_sh.at[pl.ds(r0, _RPT)])
    pltpu.sync_copy(z_hbm.at[pl.ds(r0, _RPT)], agg_sh.at[pl.ds(r0, _RPT)])
    plsc.subcore_barrier()

    chunk0 = w * _CPW + jnp.minimum(w, _XTRA)
    tail = _CPW - _FW * _WCH + jnp.where(w < _XTRA, 1, 0)

    def do_chunks(j0, n2, roff):
      pltpu.async_copy(x_sh.at[sidx.at[roff + j0]], rb0, gs0)

      def pair(k, c):
        j = roff + j0 + 2 * k
        pltpu.make_async_copy(x_sh.at[sidx.at[j]], rb0, gs0).wait()
        pltpu.async_copy(x_sh.at[sidx.at[j + 1]], rb1, gs1)
        pltpu.sync_copy(rb0, agg_sh.at[didx.at[j]], add=True)
        pltpu.make_async_copy(x_sh.at[sidx.at[j + 1]], rb1, gs1).wait()

        @pl.when(k < n2 - 1)
        def _():
          pltpu.async_copy(x_sh.at[sidx.at[j + 2]], rb0, gs0)

        pltpu.sync_copy(rb1, agg_sh.at[didx.at[j + 1]], add=True)
        return c

      lax.fori_loop(0, n2, pair, 0)

    pltpu.sync_copy(src_hbm.at[pl.ds(chunk0, _WCH)], sidx.at[pl.ds(0, _WCH)])
    pltpu.sync_copy(dst_hbm.at[pl.ds(chunk0, _WCH)], didx.at[pl.ds(0, _WCH)])

    def window(win, carry):
      p = lax.rem(win, 2)
      roff = p * _WCH
      nroff = (1 - p) * _WCH
      nbase = chunk0 + (win + 1) * _WCH

      @pl.when(win < _FW - 1)
      def _():
        pltpu.async_copy(src_hbm.at[pl.ds(nbase, _WCH)],
                         sidx.at[pl.ds(nroff, _WCH)], isem)
        pltpu.async_copy(dst_hbm.at[pl.ds(nbase, _WCH)],
                         didx.at[pl.ds(nroff, _WCH)], isem)

      do_chunks(0, _WCH // 2, roff)

      @pl.when(win < _FW - 1)
      def _():
        pltpu.make_async_copy(src_hbm.at[pl.ds(nbase, _WCH)],
                              sidx.at[pl.ds(nroff, _WCH)], isem).wait()
        pltpu.make_async_copy(dst_hbm.at[pl.ds(nbase, _WCH)],
                              didx.at[pl.ds(nroff, _WCH)], isem).wait()
      return carry

    lax.fori_loop(0, _FW, window, 0)

    tbase = chunk0 + _FW * _WCH

    @pl.when(w < _XTRA)
    def _():
      pltpu.sync_copy(src_hbm.at[pl.ds(tbase, 11)], sidx.at[pl.ds(0, 11)])
      pltpu.sync_copy(dst_hbm.at[pl.ds(tbase, 11)], didx.at[pl.ds(0, 11)])

    @pl.when(w >= _XTRA)
    def _():
      pltpu.sync_copy(src_hbm.at[pl.ds(tbase, 10)], sidx.at[pl.ds(0, 10)])
      pltpu.sync_copy(dst_hbm.at[pl.ds(tbase, 10)], didx.at[pl.ds(0, 10)])

    def tail_chunk(j, c):
      pltpu.sync_copy(x_sh.at[sidx.at[j]], rb0)
      pltpu.sync_copy(rb0, agg_sh.at[didx.at[j]], add=True)
      return c

    lax.fori_loop(0, tail, tail_chunk, 0)

    plsc.subcore_barrier()
    pltpu.sync_copy(agg_sh.at[pl.ds(r0, _RPT)],
                    out_hbm.at[cid].at[pl.ds(r0, _RPT)])

  return seg(xp, srcp, dstp, zrows)


_BR = 368


def _mlp_body(xp_ref, pp_ref, eps_ref, g1_ref, g2_ref, h1_ref, h2_ref, h3_ref,
              b1_ref, b2_ref, c1_ref, c2_ref, c3_ref, o_ref):
  h = (1.0 + eps_ref[0]) * xp_ref[...] + pp_ref[0] + pp_ref[1]
  h = jnp.dot(h, g1_ref[...], preferred_element_type=jnp.float32)
  h = jnp.maximum(h + b1_ref[...], 0.0)
  h = jnp.dot(h, g2_ref[...], preferred_element_type=jnp.float32) + b2_ref[...]
  h = jnp.dot(h, h1_ref[...], preferred_element_type=jnp.float32) + c1_ref[...]
  h = 1.0 / (1.0 + jnp.exp(-h))
  h = jnp.dot(h, h2_ref[...], preferred_element_type=jnp.float32) + c2_ref[...]
  h = 1.0 / (1.0 + jnp.exp(-h))
  h = jnp.dot(h, h3_ref[...], preferred_element_type=jnp.float32) + c3_ref[...]
  o_ref[...] = 1.0 / (1.0 + jnp.exp(-h))


def _tc_mlp_packed(xp128, pp128, eps, gin_W1, gin_b1, gin_W2, gin_b2, W1, b1,
                   W2, b2, W3, b3):
  eye = jnp.eye(_PK, dtype=jnp.float32)
  wg1 = jnp.zeros((_FP, 20), jnp.float32).at[:_F].set(gin_W1)
  g1 = jnp.kron(eye, wg1)
  g2 = jnp.kron(eye, gin_W2)
  h1 = jnp.kron(eye, W1)
  h2 = jnp.kron(eye, W2)
  h3 = jnp.kron(eye, W3)
  b1t = jnp.tile(gin_b1, _PK).reshape(1, -1)
  b2t = jnp.tile(gin_b2, _PK).reshape(1, -1)
  c1t = jnp.tile(b1, _PK).reshape(1, -1)
  c2t = jnp.tile(b2, _PK).reshape(1, -1)
  c3t = jnp.tile(b3, _PK).reshape(1, -1)
  eps1 = eps.reshape(1)

  grid = (_NR // _BR,)

  def full(a):
    return pl.BlockSpec(a.shape, lambda i: tuple(0 for _ in a.shape))

  return pl.pallas_call(
      _mlp_body,
      grid=grid,
      in_specs=[
          pl.BlockSpec((_BR, 128), lambda i: (i, 0)),
          pl.BlockSpec((2, _BR, 128), lambda i: (0, i, 0)),
          pl.BlockSpec(memory_space=pltpu.SMEM),
          full(g1), full(g2), full(h1), full(h2), full(h3),
          full(b1t), full(b2t), full(c1t), full(c2t), full(c3t),
      ],
      out_specs=pl.BlockSpec((_BR, _PK), lambda i: (i, 0)),
      out_shape=jax.ShapeDtypeStruct((_NR, _PK), jnp.float32),
  )(xp128, pp128, eps1, g1, g2, h1, h2, h3, b1t, b2t, c1t, c2t, c3t)


def kernel(x, edge_index, eps, gin_W1, gin_b1, gin_W2, gin_b2, W1, b1, W2, b2,
           W3, b3):
  srcp = edge_index[0].reshape(_NCH, _CHUNK)
  dstp = edge_index[1].reshape(_NCH, _CHUNK)
  xp = jnp.zeros((_NP, _FP), jnp.float32).at[:_N, :_F].set(x)
  zrows = jnp.zeros((_NP, _FP), jnp.float32)
  partials = _sc_segment_sum(xp, srcp, dstp, zrows)
  out_packed = _tc_mlp_packed(xp.reshape(_NR, 128),
                              partials.reshape(_NC, _NR, 128), eps,
                              gin_W1, gin_b1, gin_W2, gin_b2, W1, b1, W2, b2,
                              W3, b3)
  return out_packed.reshape(_NP, 1)[:_N]


# speedup vs baseline: 128.8279x; 1.2951x over previous
import functools

import jax
import jax.numpy as jnp
from jax import lax
from jax.experimental import pallas as pl
from jax.experimental.pallas import tpu as pltpu
from jax.experimental.pallas import tpu_sc as plsc

_N = 100000
_E = 6400000
_F = 4
_FP = 8
_NC = 2
_NS = 16
_NW = _NC * _NS
_CHUNK = 512
_WCH = 8
_NCH = _E // _CHUNK
_CPW = _NCH // _NW
_XTRA = _NCH % _NW
_FW = _CPW // _WCH
_TAIL = _CPW - _FW * _WCH
_NP = _N + 96
_RPT = _NP // _NS
_PK = 128 // _FP
_NR = _NP // _PK


def _sc_segment_sum(xp_rows, srcp, dstp, zrows):
  mesh = plsc.VectorSubcoreMesh(core_axis_name="c", subcore_axis_name="s")

  @functools.partial(
      pl.kernel,
      out_type=jax.ShapeDtypeStruct((_NC, _NP, _FP), jnp.float32),
      mesh=mesh,
      compiler_params=pltpu.CompilerParams(use_tc_tiling_on_sc=False),
      scratch_types=[
          pltpu.VMEM_SHARED((_NP, _FP), jnp.float32),
          pltpu.VMEM_SHARED((_NP, _FP), jnp.float32),
          pltpu.VMEM((2 * _WCH, _CHUNK), jnp.int32),
          pltpu.VMEM((2 * _WCH, _CHUNK), jnp.int32),
          pltpu.VMEM((_CHUNK, _FP), jnp.float32),
          pltpu.VMEM((_CHUNK, _FP), jnp.float32),
          pltpu.SemaphoreType.DMA,
          pltpu.SemaphoreType.DMA,
          pltpu.SemaphoreType.DMA,
      ],
  )
  def seg(x_hbm, src_hbm, dst_hbm, z_hbm, out_hbm, x_sh, agg_sh, sidx, didx,
          rb0, rb1, gs0, gs1, isem):
    cid = lax.axis_index("c")
    sid = lax.axis_index("s")
    w = cid * _NS + sid
    r0 = sid * _RPT
    pltpu.sync_copy(x_hbm.at[pl.ds(r0, _RPT)], x_sh.at[pl.ds(r0, _RPT)])
    pltpu.sync_copy(z_hbm.at[pl.ds(r0, _RPT)], agg_sh.at[pl.ds(r0, _RPT)])
    plsc.subcore_barrier()

    chunk0 = w * _CPW + jnp.minimum(w, _XTRA)
    tail = _TAIL + jnp.where(w < _XTRA, 1, 0)

    def do_chunks(n2, roff):
      pltpu.async_copy(x_sh.at[sidx.at[roff]], rb0, gs0)

      def pair(k, c):
        j = roff + 2 * k
        pltpu.make_async_copy(x_sh.at[sidx.at[j]], rb0, gs0).wait()
        pltpu.async_copy(x_sh.at[sidx.at[j + 1]], rb1, gs1)
        pltpu.sync_copy(rb0, agg_sh.at[didx.at[j]], add=True)
        pltpu.make_async_copy(x_sh.at[sidx.at[j + 1]], rb1, gs1).wait()

        @pl.when(k < n2 - 1)
        def _():
          pltpu.async_copy(x_sh.at[sidx.at[j + 2]], rb0, gs0)

        pltpu.sync_copy(rb1, agg_sh.at[didx.at[j + 1]], add=True)
        return c

      lax.fori_loop(0, n2, pair, 0)

    pltpu.sync_copy(src_hbm.at[pl.ds(chunk0, _WCH)], sidx.at[pl.ds(0, _WCH)])
    pltpu.sync_copy(dst_hbm.at[pl.ds(chunk0, _WCH)], didx.at[pl.ds(0, _WCH)])

    def window(win, carry):
      p = lax.rem(win, 2)
      roff = p * _WCH
      nroff = (1 - p) * _WCH
      nbase = chunk0 + (win + 1) * _WCH

      @pl.when(win < _FW - 1)
      def _():
        pltpu.async_copy(src_hbm.at[pl.ds(nbase, _WCH)],
                         sidx.at[pl.ds(nroff, _WCH)], isem)
        pltpu.async_copy(dst_hbm.at[pl.ds(nbase, _WCH)],
                         didx.at[pl.ds(nroff, _WCH)], isem)

      do_chunks(_WCH // 2, roff)

      @pl.when(win < _FW - 1)
      def _():
        pltpu.make_async_copy(src_hbm.at[pl.ds(nbase, _WCH)],
                              sidx.at[pl.ds(nroff, _WCH)], isem).wait()
        pltpu.make_async_copy(dst_hbm.at[pl.ds(nbase, _WCH)],
                              didx.at[pl.ds(nroff, _WCH)], isem).wait()
      return carry

    lax.fori_loop(0, _FW, window, 0)

    tbase = chunk0 + _FW * _WCH

    @pl.when(w < _XTRA)
    def _():
      pltpu.sync_copy(src_hbm.at[pl.ds(tbase, _TAIL + 1)],
                      sidx.at[pl.ds(0, _TAIL + 1)])
      pltpu.sync_copy(dst_hbm.at[pl.ds(tbase, _TAIL + 1)],
                      didx.at[pl.ds(0, _TAIL + 1)])

    @pl.when(w >= _XTRA)
    def _():
      pltpu.sync_copy(src_hbm.at[pl.ds(tbase, _TAIL)],
                      sidx.at[pl.ds(0, _TAIL)])
      pltpu.sync_copy(dst_hbm.at[pl.ds(tbase, _TAIL)],
                      didx.at[pl.ds(0, _TAIL)])

    def tail_chunk(j, c):
      pltpu.sync_copy(x_sh.at[sidx.at[j]], rb0)
      pltpu.sync_copy(rb0, agg_sh.at[didx.at[j]], add=True)
      return c

    lax.fori_loop(0, tail, tail_chunk, 0)

    plsc.subcore_barrier()
    pltpu.sync_copy(agg_sh.at[pl.ds(r0, _RPT)],
                    out_hbm.at[cid].at[pl.ds(r0, _RPT)])

  return seg(xp_rows, srcp, dstp, zrows)


_BR = 368


def _mlp_body(xp_ref, pp_ref, eps_ref, g1_ref, g2_ref, h1_ref, h2_ref, h3_ref,
              b1_ref, b2_ref, c1_ref, c2_ref, c3_ref, o_ref):
  h = (1.0 + eps_ref[0]) * xp_ref[...] + pp_ref[0] + pp_ref[1]
  h = jnp.dot(h, g1_ref[...], preferred_element_type=jnp.float32)
  h = jnp.maximum(h + b1_ref[...], 0.0)
  h = jnp.dot(h, g2_ref[...], preferred_element_type=jnp.float32) + b2_ref[...]
  h = jnp.dot(h, h1_ref[...], preferred_element_type=jnp.float32) + c1_ref[...]
  h = 1.0 / (1.0 + jnp.exp(-h))
  h = jnp.dot(h, h2_ref[...], preferred_element_type=jnp.float32) + c2_ref[...]
  h = 1.0 / (1.0 + jnp.exp(-h))
  h = jnp.dot(h, h3_ref[...], preferred_element_type=jnp.float32) + c3_ref[...]
  o_ref[...] = 1.0 / (1.0 + jnp.exp(-h))


def _tc_mlp_packed(xp128, pp128, eps, gin_W1, gin_b1, gin_W2, gin_b2, W1, b1,
                   W2, b2, W3, b3):
  eye = jnp.eye(_PK, dtype=jnp.float32)
  wg1 = jnp.zeros((_FP, 20), jnp.float32).at[:_F].set(gin_W1)
  g1 = jnp.kron(eye, wg1)
  g2 = jnp.kron(eye, gin_W2)
  h1 = jnp.kron(eye, W1)
  h2 = jnp.kron(eye, W2)
  h3 = jnp.kron(eye, W3)
  b1t = jnp.tile(gin_b1, _PK).reshape(1, -1)
  b2t = jnp.tile(gin_b2, _PK).reshape(1, -1)
  c1t = jnp.tile(b1, _PK).reshape(1, -1)
  c2t = jnp.tile(b2, _PK).reshape(1, -1)
  c3t = jnp.tile(b3, _PK).reshape(1, -1)
  eps1 = eps.reshape(1)
  grid = (_NR // _BR,)

  def full(a):
    return pl.BlockSpec(a.shape, lambda i: tuple(0 for _ in a.shape))

  return pl.pallas_call(
      _mlp_body,
      grid=grid,
      in_specs=[
          pl.BlockSpec((_BR, 128), lambda i: (i, 0)),
          pl.BlockSpec((2, _BR, 128), lambda i: (0, i, 0)),
          pl.BlockSpec(memory_space=pltpu.SMEM),
          full(g1), full(g2), full(h1), full(h2), full(h3),
          full(b1t), full(b2t), full(c1t), full(c2t), full(c3t),
      ],
      out_specs=pl.BlockSpec((_BR, _PK), lambda i: (i, 0)),
      out_shape=jax.ShapeDtypeStruct((_NR, _PK), jnp.float32),
  )(xp128, pp128, eps1, g1, g2, h1, h2, h3, b1t, b2t, c1t, c2t, c3t)


def kernel(x, edge_index, eps, gin_W1, gin_b1, gin_W2, gin_b2, W1, b1, W2, b2,
           W3, b3):
  srcp = edge_index[0].reshape(_NCH, _CHUNK)
  dstp = edge_index[1].reshape(_NCH, _CHUNK)
  s_mat = jnp.kron(jnp.eye(_PK, dtype=jnp.float32),
                   jnp.zeros((_F, _FP), jnp.float32).at[:, :_F].set(
                       jnp.eye(_F, dtype=jnp.float32)))
  xp128 = jnp.pad(x.reshape(_N // _PK, _F * _PK) @ s_mat,
                  ((0, _NR - _N // _PK), (0, 0)))
  zrows = jnp.zeros((_NP, _FP), jnp.float32)
  partials = _sc_segment_sum(xp128.reshape(_NP, _FP), srcp, dstp, zrows)
  out_packed = _tc_mlp_packed(xp128, partials.reshape(_NC, _NR, 128), eps,
                              gin_W1, gin_b1, gin_W2, gin_b2, W1, b1, W2, b2,
                              W3, b3)
  return out_packed.reshape(_NP, 1)[:_N]
